# baseline (speedup 1.0000x reference)
import functools

import jax
import jax.numpy as jnp
from jax import lax
from jax.experimental import pallas as pl
from jax.experimental.pallas import tpu as pltpu
from jax.experimental.pallas import tpu_sc as plsc

E = 8
K = 2
D = 768
L = 2048
H = 4 * D

A = K * L
TB = 128
G = (A + E * (TB - 1)) // TB + 1
P = G * TB
GPAD = 64

NW = 32
RPW = P // NW
RC = 40
TPW = L // NW
CB = 16


def _gelu_exact(h):
    return 0.5 * h * (1.0 + lax.erf(h * (2.0 ** -0.5)))


def _excl_cumsum_cols(m, chunk=256):
    n = m.shape[0]
    ri = lax.broadcasted_iota(jnp.int32, (chunk, chunk), 0)
    ci = lax.broadcasted_iota(jnp.int32, (chunk, chunk), 1)
    tstrict = (ci < ri).astype(jnp.float32)
    carry = jnp.zeros((1, m.shape[1]), jnp.float32)
    parts = []
    for c in range(n // chunk):
        blk = m[c * chunk:(c + 1) * chunk, :]
        parts.append(jnp.dot(tstrict, blk, preferred_element_type=jnp.float32)
                     + carry)
        carry = carry + jnp.sum(blk, axis=0, keepdims=True)
    return jnp.concatenate(parts, axis=0)


def _router_body(x_ref, wg_ref, pos_ref, g_ref, be_ref, loss_ref):
    x = x_ref[...]
    wg = wg_ref[...]
    logits = jnp.dot(x, wg, preferred_element_type=jnp.float32)
    lane = lax.broadcasted_iota(jnp.int32, (L, E), 1)
    m1 = jnp.max(logits, axis=1, keepdims=True)
    i1 = jnp.min(jnp.where(logits == m1, lane, E), axis=1, keepdims=True)
    oh1 = (lane == i1).astype(jnp.float32)
    logits2 = jnp.where(lane == i1, -jnp.inf, logits)
    m2 = jnp.max(logits2, axis=1, keepdims=True)
    i2 = jnp.min(jnp.where(logits2 == m2, lane, E), axis=1, keepdims=True)
    oh2 = (lane == i2).astype(jnp.float32)
    a = jnp.exp(m2 - m1)
    g1 = 1.0 / (1.0 + a)
    g2 = a / (1.0 + a)
    g_ref[...] = jnp.concatenate([g1, g2], axis=1)

    counts = jnp.sum(oh1 + oh2, axis=0, keepdims=True)
    cn = counts / A
    loss_ref[...] = jnp.sum((cn - 1.0 / E) ** 2, axis=1, keepdims=True) / E

    pc = jnp.floor((counts + (TB - 1)) / TB) * TB
    ei = lax.broadcasted_iota(jnp.int32, (E, E), 0)
    ej = lax.broadcasted_iota(jnp.int32, (E, E), 1)
    te = (ei < ej).astype(jnp.float32)
    off = jnp.dot(pc, te, preferred_element_type=jnp.float32)
    end = off + pc

    cs = _excl_cumsum_cols(oh1 + oh2)
    p0 = jnp.sum(oh1 * (off + cs), axis=1, keepdims=True)
    p1 = jnp.sum(oh2 * (off + cs + oh1), axis=1, keepdims=True)
    pos_ref[...] = jnp.concatenate([p0, p1], axis=1).astype(jnp.int32)

    b128 = lax.broadcasted_iota(jnp.int32, (GPAD, E), 0).astype(jnp.float32) * TB
    bc = jnp.sum((b128 >= end).astype(jnp.float32), axis=1, keepdims=True)
    be_ref[...] = jnp.minimum(bc, E - 1).astype(jnp.int32)


def _dispatch_body(pos_hbm, g_hbm, st_hbm, sg_hbm,
                   pos_v, g_v, st_v, sg_v):
    wid = lax.axis_index("s") * 2 + lax.axis_index("c")

    @pl.when(wid == 0)
    def _():
        with jax.named_scope("disp_load"):
            pltpu.sync_copy(pos_hbm, pos_v)
            pltpu.sync_copy(g_hbm, g_v)

        zi = jnp.zeros((16,), jnp.int32)
        zf = jnp.zeros((16,), jnp.float32)

        with jax.named_scope("disp_init"):
            def init_body(i, carry):
                for u in range(4):
                    st_v[pl.ds(i * 64 + u * 16, 16)] = zi
                    sg_v[pl.ds(i * 64 + u * 16, 16)] = zf
                return carry

            lax.fori_loop(0, P // 64, init_body, 0)

        iota16 = lax.iota(jnp.int32, 16)

        with jax.named_scope("disp_scatter"):
            def scat_body(i, carry):
                for u in range(4):
                    o = i * 64 + u * 16
                    idx = pos_v[pl.ds(o, 16)]
                    tok = lax.shift_right_logical(o + iota16, 1)
                    plsc.store_scatter(st_v, [idx], tok)
                    plsc.store_scatter(sg_v, [idx], g_v[pl.ds(o, 16)])
                return carry

            lax.fori_loop(0, A // 64, scat_body, 0)

        with jax.named_scope("disp_write"):
            pltpu.sync_copy(st_v, st_hbm)
            pltpu.sync_copy(sg_v, sg_hbm)


def _gmm_body(be_ref, st_ref, x_ref, w1_ref, w2_ref, sg_ref, ys_ref):
    stb = st_ref[...]
    tok = lax.broadcasted_iota(jnp.int32, (TB, L), 1)
    oh = (tok == stb).astype(jnp.bfloat16)
    xb = jnp.dot(oh, x_ref[...], preferred_element_type=jnp.float32)
    xb = xb.astype(jnp.bfloat16)
    h = _gelu_exact(jnp.dot(xb, w1_ref[0], preferred_element_type=jnp.float32))
    y = jnp.dot(h.astype(jnp.bfloat16), w2_ref[0],
                preferred_element_type=jnp.float32)
    ys_ref[...] = y * sg_ref[...]


def _combine_body(ys_hbm, pos_hbm, out_hbm,
                  p_v, b0, b1, o0, o1, gs0, gs1, ws0, ws1):
    wid = lax.axis_index("s") * 2 + lax.axis_index("c")
    tbase = wid * TPW
    pltpu.sync_copy(pos_hbm.at[pl.ds(2 * tbase, 2 * TPW)], p_v)

    def gat(c, buf, sem):
        return pltpu.async_copy(
            ys_hbm.at[p_v.at[pl.ds(c * 2 * CB, 2 * CB)]], buf, sem)

    def wrt(c, buf, sem):
        return pltpu.async_copy(
            buf, out_hbm.at[pl.ds(tbase + c * CB, CB)], sem)

    def add(buf, obuf):
        def row_body(r, rc):
            for k in range(D // 16):
                s = pl.ds(k * 16, 16)
                obuf[r, s] = buf[2 * r, s] + buf[2 * r + 1, s]
            return rc

        lax.fori_loop(0, CB, row_body, 0)

    cg0 = gat(0, b0, gs0)
    cg1 = gat(1, b1, gs1)
    cg0.wait()
    add(b0, o0)
    cg2 = gat(2, b0, gs0)
    cw0 = wrt(0, o0, ws0)
    cg1.wait()
    add(b1, o1)
    cg3 = gat(3, b1, gs1)
    cw1 = wrt(1, o1, ws1)
    cg2.wait()
    cw0.wait()
    add(b0, o0)
    cw2 = wrt(2, o0, ws0)
    cg3.wait()
    cw1.wait()
    add(b1, o1)
    cw3 = wrt(3, o1, ws1)
    cw2.wait()
    cw3.wait()


def _make_sc_kernels():
    mesh = plsc.VectorSubcoreMesh(core_axis_name="c", subcore_axis_name="s",
                                  num_cores=2, num_subcores=16)
    sc_params = pltpu.CompilerParams(needs_layout_passes=False)
    dispatch = pl.kernel(
        _dispatch_body,
        compiler_params=sc_params,
        out_type=(
            jax.ShapeDtypeStruct((P,), jnp.int32),
            jax.ShapeDtypeStruct((P,), jnp.float32),
        ),
        mesh=mesh,
        scratch_types=[
            pltpu.VMEM((A,), jnp.int32),
            pltpu.VMEM((A,), jnp.float32),
            pltpu.VMEM((P,), jnp.int32),
            pltpu.VMEM((P,), jnp.float32),
        ],
    )
    combine = pl.kernel(
        _combine_body,
        compiler_params=sc_params,
        out_type=jax.ShapeDtypeStruct((L, D), jnp.float32),
        mesh=mesh,
        scratch_types=[
            pltpu.VMEM((2 * TPW,), jnp.int32),
            pltpu.VMEM((2 * CB, D), jnp.float32),
            pltpu.VMEM((2 * CB, D), jnp.float32),
            pltpu.VMEM((CB, D), jnp.float32),
            pltpu.VMEM((CB, D), jnp.float32),
            pltpu.SemaphoreType.DMA,
            pltpu.SemaphoreType.DMA,
            pltpu.SemaphoreType.DMA,
            pltpu.SemaphoreType.DMA,
        ],
    )
    return dispatch, combine


def kernel(x, Wg, W1, W2):
    _dispatch, _combine = _make_sc_kernels()
    x2 = x.reshape(L, D)

    pos, g01, be, loss = pl.pallas_call(
        _router_body,
        out_shape=(
            jax.ShapeDtypeStruct((L, K), jnp.int32),
            jax.ShapeDtypeStruct((L, K), jnp.float32),
            jax.ShapeDtypeStruct((GPAD, 1), jnp.int32),
            jax.ShapeDtypeStruct((1, 1), jnp.float32),
        ),
    )(x2, Wg)

    st, sg = _dispatch(pos.reshape(A), g01.reshape(A))

    grid_spec = pltpu.PrefetchScalarGridSpec(
        num_scalar_prefetch=1,
        grid=(G,),
        in_specs=[
            pl.BlockSpec((TB, 1), lambda b, be: (b, 0)),
            pl.BlockSpec((L, D), lambda b, be: (0, 0)),
            pl.BlockSpec((1, D, H), lambda b, be: (be[b], 0, 0)),
            pl.BlockSpec((1, H, D), lambda b, be: (be[b], 0, 0)),
            pl.BlockSpec((TB, 1), lambda b, be: (b, 0)),
        ],
        out_specs=pl.BlockSpec((TB, D), lambda b, be: (b, 0)),
    )
    ys = pl.pallas_call(
        _gmm_body,
        grid_spec=grid_spec,
        out_shape=jax.ShapeDtypeStruct((P, D), jnp.float32),
    )(be.reshape(GPAD)[:G], st.reshape(P, 1), x2.astype(jnp.bfloat16),
      W1.astype(jnp.bfloat16), W2.astype(jnp.bfloat16), sg.reshape(P, 1))

    out = _combine(ys, pos.reshape(A))

    return out.reshape(1, L, D), loss.reshape(())


# speedup vs baseline: 1.2534x; 1.2534x over previous
import functools

import jax
import jax.numpy as jnp
from jax import lax
from jax.experimental import pallas as pl
from jax.experimental.pallas import tpu as pltpu
from jax.experimental.pallas import tpu_sc as plsc

E = 8
K = 2
D = 768
L = 2048
H = 4 * D

A = K * L
TB = 256
G = (A + E * (TB - 1)) // TB + 1
P = G * TB
GPAD = 64

NW = 32
RPW = P // NW
RC = 40
TPW = L // NW
CB = 16


def _gelu_exact(h):
    return 0.5 * h * (1.0 + lax.erf(h * (2.0 ** -0.5)))


def _excl_cumsum_cols(m, chunk=256):
    n = m.shape[0]
    ri = lax.broadcasted_iota(jnp.int32, (chunk, chunk), 0)
    ci = lax.broadcasted_iota(jnp.int32, (chunk, chunk), 1)
    tstrict = (ci < ri).astype(jnp.float32)
    carry = jnp.zeros((1, m.shape[1]), jnp.float32)
    parts = []
    for c in range(n // chunk):
        blk = m[c * chunk:(c + 1) * chunk, :]
        parts.append(jnp.dot(tstrict, blk, preferred_element_type=jnp.float32)
                     + carry)
        carry = carry + jnp.sum(blk, axis=0, keepdims=True)
    return jnp.concatenate(parts, axis=0)


def _router_body(x_ref, wg_ref, pos_ref, g_ref, be_ref, loss_ref):
    x = x_ref[...]
    wg = wg_ref[...]
    logits = jnp.dot(x, wg, preferred_element_type=jnp.float32)
    lane = lax.broadcasted_iota(jnp.int32, (L, E), 1)
    m1 = jnp.max(logits, axis=1, keepdims=True)
    i1 = jnp.min(jnp.where(logits == m1, lane, E), axis=1, keepdims=True)
    oh1 = (lane == i1).astype(jnp.float32)
    logits2 = jnp.where(lane == i1, -jnp.inf, logits)
    m2 = jnp.max(logits2, axis=1, keepdims=True)
    i2 = jnp.min(jnp.where(logits2 == m2, lane, E), axis=1, keepdims=True)
    oh2 = (lane == i2).astype(jnp.float32)
    a = jnp.exp(m2 - m1)
    g1 = 1.0 / (1.0 + a)
    g2 = a / (1.0 + a)
    g_ref[...] = jnp.concatenate([g1, g2], axis=1)

    counts = jnp.sum(oh1 + oh2, axis=0, keepdims=True)
    cn = counts / A
    loss_ref[...] = jnp.sum((cn - 1.0 / E) ** 2, axis=1, keepdims=True) / E

    pc = jnp.floor((counts + (TB - 1)) / TB) * TB
    ei = lax.broadcasted_iota(jnp.int32, (E, E), 0)
    ej = lax.broadcasted_iota(jnp.int32, (E, E), 1)
    te = (ei < ej).astype(jnp.float32)
    off = jnp.dot(pc, te, preferred_element_type=jnp.float32)
    end = off + pc

    cs = _excl_cumsum_cols(oh1 + oh2)
    p0 = jnp.sum(oh1 * (off + cs), axis=1, keepdims=True)
    p1 = jnp.sum(oh2 * (off + cs + oh1), axis=1, keepdims=True)
    pos_ref[...] = jnp.concatenate([p0, p1], axis=1).astype(jnp.int32)

    b128 = lax.broadcasted_iota(jnp.int32, (GPAD, E), 0).astype(jnp.float32) * TB
    bc = jnp.sum((b128 >= end).astype(jnp.float32), axis=1, keepdims=True)
    be_ref[...] = jnp.minimum(bc, E - 1).astype(jnp.int32)


def _dispatch_body(pos_hbm, g_hbm, st_hbm, sg_hbm,
                   pos_v, g_v, st_v, sg_v):
    wid = lax.axis_index("s") * 2 + lax.axis_index("c")

    @pl.when(wid == 0)
    def _():
        with jax.named_scope("disp_load"):
            pltpu.sync_copy(pos_hbm, pos_v)
            pltpu.sync_copy(g_hbm, g_v)

        zi = jnp.zeros((16,), jnp.int32)
        zf = jnp.zeros((16,), jnp.float32)

        with jax.named_scope("disp_init"):
            def init_body(i, carry):
                for u in range(4):
                    st_v[pl.ds(i * 64 + u * 16, 16)] = zi
                    sg_v[pl.ds(i * 64 + u * 16, 16)] = zf
                return carry

            lax.fori_loop(0, P // 64, init_body, 0)

        iota16 = lax.iota(jnp.int32, 16)

        with jax.named_scope("disp_scatter"):
            def scat_body(i, carry):
                for u in range(4):
                    o = i * 64 + u * 16
                    idx = pos_v[pl.ds(o, 16)]
                    tok = lax.shift_right_logical(o + iota16, 1)
                    plsc.store_scatter(st_v, [idx], tok)
                    plsc.store_scatter(sg_v, [idx], g_v[pl.ds(o, 16)])
                return carry

            lax.fori_loop(0, A // 64, scat_body, 0)

        with jax.named_scope("disp_write"):
            pltpu.sync_copy(st_v, st_hbm)
            pltpu.sync_copy(sg_v, sg_hbm)


def _gmm_body(be_ref, st_ref, x_ref, w1_ref, w2_ref, sg_ref, ys_ref):
    stb = st_ref[...]
    tok = lax.broadcasted_iota(jnp.int32, (TB, L), 1)
    oh = (tok == stb).astype(jnp.float32)
    xb = jnp.dot(oh, x_ref[...], preferred_element_type=jnp.float32)
    h = _gelu_exact(jnp.dot(xb, w1_ref[0], preferred_element_type=jnp.float32))
    y = jnp.dot(h, w2_ref[0], preferred_element_type=jnp.float32)
    ys_ref[...] = y * sg_ref[...]


def _combine_body(ys_hbm, pos_hbm, out_hbm,
                  p_v, b0, b1, o0, o1, gs0, gs1, ws0, ws1):
    wid = lax.axis_index("s") * 2 + lax.axis_index("c")
    tbase = wid * TPW
    pltpu.sync_copy(pos_hbm.at[pl.ds(2 * tbase, 2 * TPW)], p_v)

    def gat(c, buf, sem):
        return pltpu.async_copy(
            ys_hbm.at[p_v.at[pl.ds(c * 2 * CB, 2 * CB)]], buf, sem)

    def wrt(c, buf, sem):
        return pltpu.async_copy(
            buf, out_hbm.at[pl.ds(tbase + c * CB, CB)], sem)

    def add(buf, obuf):
        def row_body(r, rc):
            for k in range(D // 16):
                s = pl.ds(k * 16, 16)
                obuf[r, s] = buf[2 * r, s] + buf[2 * r + 1, s]
            return rc

        lax.fori_loop(0, CB, row_body, 0)

    cg0 = gat(0, b0, gs0)
    cg1 = gat(1, b1, gs1)
    cg0.wait()
    add(b0, o0)
    cg2 = gat(2, b0, gs0)
    cw0 = wrt(0, o0, ws0)
    cg1.wait()
    add(b1, o1)
    cg3 = gat(3, b1, gs1)
    cw1 = wrt(1, o1, ws1)
    cg2.wait()
    cw0.wait()
    add(b0, o0)
    cw2 = wrt(2, o0, ws0)
    cg3.wait()
    cw1.wait()
    add(b1, o1)
    cw3 = wrt(3, o1, ws1)
    cw2.wait()
    cw3.wait()


def _make_sc_kernels():
    mesh = plsc.VectorSubcoreMesh(core_axis_name="c", subcore_axis_name="s",
                                  num_cores=2, num_subcores=16)
    sc_params = pltpu.CompilerParams(needs_layout_passes=False)
    dispatch = pl.kernel(
        _dispatch_body,
        compiler_params=sc_params,
        out_type=(
            jax.ShapeDtypeStruct((P,), jnp.int32),
            jax.ShapeDtypeStruct((P,), jnp.float32),
        ),
        mesh=mesh,
        scratch_types=[
            pltpu.VMEM((A,), jnp.int32),
            pltpu.VMEM((A,), jnp.float32),
            pltpu.VMEM((P,), jnp.int32),
            pltpu.VMEM((P,), jnp.float32),
        ],
    )
    combine = pl.kernel(
        _combine_body,
        compiler_params=sc_params,
        out_type=jax.ShapeDtypeStruct((L, D), jnp.float32),
        mesh=mesh,
        scratch_types=[
            pltpu.VMEM((2 * TPW,), jnp.int32),
            pltpu.VMEM((2 * CB, D), jnp.float32),
            pltpu.VMEM((2 * CB, D), jnp.float32),
            pltpu.VMEM((CB, D), jnp.float32),
            pltpu.VMEM((CB, D), jnp.float32),
            pltpu.SemaphoreType.DMA,
            pltpu.SemaphoreType.DMA,
            pltpu.SemaphoreType.DMA,
            pltpu.SemaphoreType.DMA,
        ],
    )
    return dispatch, combine


def kernel(x, Wg, W1, W2):
    _dispatch, _combine = _make_sc_kernels()
    x2 = x.reshape(L, D)

    pos, g01, be, loss = pl.pallas_call(
        _router_body,
        out_shape=(
            jax.ShapeDtypeStruct((L, K), jnp.int32),
            jax.ShapeDtypeStruct((L, K), jnp.float32),
            jax.ShapeDtypeStruct((GPAD, 1), jnp.int32),
            jax.ShapeDtypeStruct((1, 1), jnp.float32),
        ),
    )(x2, Wg)

    st, sg = _dispatch(pos.reshape(A), g01.reshape(A))

    grid_spec = pltpu.PrefetchScalarGridSpec(
        num_scalar_prefetch=1,
        grid=(G,),
        in_specs=[
            pl.BlockSpec((TB, 1), lambda b, be: (b, 0)),
            pl.BlockSpec((L, D), lambda b, be: (0, 0)),
            pl.BlockSpec((1, D, H), lambda b, be: (be[b], 0, 0)),
            pl.BlockSpec((1, H, D), lambda b, be: (be[b], 0, 0)),
            pl.BlockSpec((TB, 1), lambda b, be: (b, 0)),
        ],
        out_specs=pl.BlockSpec((TB, D), lambda b, be: (b, 0)),
    )
    ys = pl.pallas_call(
        _gmm_body,
        grid_spec=grid_spec,
        out_shape=jax.ShapeDtypeStruct((P, D), jnp.float32),
    )(be.reshape(GPAD)[:G], st.reshape(P, 1), x2, W1, W2, sg.reshape(P, 1))

    out = _combine(ys, pos.reshape(A))

    return out.reshape(1, L, D), loss.reshape(())
